# baseline (speedup 1.0000x reference)
import jax
import jax.numpy as jnp
from jax.experimental import pallas as pl
from jax.experimental.pallas import tpu as pltpu

DIM = 128
WIDE = 8192
BLOCK_ROWS = 64


def _copy_body(x_ref, o_ref):
    o_ref[...] = x_ref[...]


def kernel(x):
    x = x.astype(jnp.float32)
    n = x.size // DIM
    wide_rows = x.size // WIDE
    xw = x.reshape(wide_rows, WIDE)
    grid = (wide_rows // BLOCK_ROWS,)
    out = pl.pallas_call(
        _copy_body,
        out_shape=jax.ShapeDtypeStruct((wide_rows, WIDE), jnp.float32),
        grid=grid,
        in_specs=[pl.BlockSpec((BLOCK_ROWS, WIDE), lambda i: (i, 0))],
        out_specs=pl.BlockSpec((BLOCK_ROWS, WIDE), lambda i: (i, 0)),
    )(xw)
    return out.reshape(n, DIM)


# speedup vs baseline: 4.0055x; 4.0055x over previous
import jax
import jax.numpy as jnp
from jax.experimental import pallas as pl
from jax.experimental.pallas import tpu as pltpu

DIM = 128
BLOCK_ROWS = 4096


def _copy_body(x_ref, o_ref):
    o_ref[...] = x_ref[...]


def kernel(x):
    x = x.astype(jnp.float32)
    n = x.size // DIM
    x = x.reshape(n, DIM)
    grid = (n // BLOCK_ROWS,)
    return pl.pallas_call(
        _copy_body,
        out_shape=jax.ShapeDtypeStruct((n, DIM), jnp.float32),
        grid=grid,
        in_specs=[pl.BlockSpec((BLOCK_ROWS, DIM), lambda i: (i, 0))],
        out_specs=pl.BlockSpec((BLOCK_ROWS, DIM), lambda i: (i, 0)),
    )(x)


# speedup vs baseline: 4.3845x; 1.0946x over previous
import jax
import jax.numpy as jnp
from jax.experimental import pallas as pl
from jax.experimental.pallas import tpu as pltpu

DIM = 128
BLOCK_ROWS = 8192


def _copy_body(x_ref, o_ref):
    o_ref[...] = x_ref[...]


def kernel(x):
    x = x.astype(jnp.float32)
    n = x.size // DIM
    x = x.reshape(n, DIM)
    grid = (n // BLOCK_ROWS,)
    return pl.pallas_call(
        _copy_body,
        out_shape=jax.ShapeDtypeStruct((n, DIM), jnp.float32),
        grid=grid,
        in_specs=[pl.BlockSpec((BLOCK_ROWS, DIM), lambda i: (i, 0))],
        out_specs=pl.BlockSpec((BLOCK_ROWS, DIM), lambda i: (i, 0)),
    )(x)


# speedup vs baseline: 4.9577x; 1.1307x over previous
import jax
import jax.numpy as jnp
from jax.experimental import pallas as pl
from jax.experimental.pallas import tpu as pltpu

DIM = 128
BLOCK_ROWS = 16384


def _copy_body(x_ref, o_ref):
    o_ref[...] = x_ref[...]


def kernel(x):
    x = x.astype(jnp.float32)
    n = x.size // DIM
    x = x.reshape(n, DIM)
    grid = (n // BLOCK_ROWS,)
    return pl.pallas_call(
        _copy_body,
        out_shape=jax.ShapeDtypeStruct((n, DIM), jnp.float32),
        grid=grid,
        in_specs=[pl.BlockSpec((BLOCK_ROWS, DIM), lambda i: (i, 0))],
        out_specs=pl.BlockSpec((BLOCK_ROWS, DIM), lambda i: (i, 0)),
    )(x)
